# baseline (speedup 1.0000x reference)
import functools

import jax
import jax.numpy as jnp
from jax import lax
from jax.experimental import pallas as pl
from jax.experimental.pallas import tpu as pltpu
from jax.experimental.pallas import tpu_sc as plsc

NUM_CODES = 1024
EMBED_DIM = 256
BETA = 0.25
BLK = 512


def _vq_body(z_ref, c_ref, zsq_ref, csq_ref, idx_ref, d2_ref):
    zb = z_ref[...]
    cb = c_ref[...]
    s = lax.dot_general(zb, cb, (((1,), (1,)), ((), ())))
    zsq = zsq_ref[0, 0, :].reshape(BLK, 1)
    csq = csq_ref[0, :].reshape(1, NUM_CODES)
    d2 = (zsq + csq) - 2.0 * s
    d2 = jnp.maximum(d2, 0.0)
    dist = jnp.sqrt(d2)
    minval = jnp.min(dist, axis=1, keepdims=True)
    iota = lax.broadcasted_iota(jnp.int32, (BLK, NUM_CODES), 1)
    idx = jnp.min(jnp.where(dist == minval, iota, jnp.int32(NUM_CODES)), axis=1)
    idx_ref[0, 0, :] = idx
    d2_ref[0, 0, :] = jnp.min(d2, axis=1)


def _sc_gather(codebook, indices):
    info = plsc.get_sparse_core_info()
    nc, ns = info.num_cores, info.num_subcores
    nw = nc * ns
    n, d = indices.shape[0], codebook.shape[1]
    bpw = n // nw
    ch = 128
    nch = bpw // ch
    idx2 = indices.reshape(n // ch, ch)
    mesh = plsc.VectorSubcoreMesh(core_axis_name="c", subcore_axis_name="s")

    @functools.partial(
        pl.kernel, mesh=mesh,
        out_type=jax.ShapeDtypeStruct((n, d), jnp.float32),
        scratch_types=[
            pltpu.VMEM((ch,), jnp.int32),
            pltpu.VMEM((ch,), jnp.int32),
            pltpu.VMEM((bpw, d), jnp.float32),
            pltpu.SemaphoreType.DMA,
        ],
    )
    def gk(table_hbm, idx_hbm, out_hbm, ia, ib, rows, sem):
        w = lax.axis_index("s") * nc + lax.axis_index("c")
        r0 = w * nch
        pltpu.sync_copy(idx_hbm.at[r0], ia)
        pltpu.sync_copy(idx_hbm.at[r0 + 1], ib)
        c1 = pltpu.async_copy(table_hbm.at[ia], rows.at[pl.ds(0, ch)], sem)
        c2 = pltpu.async_copy(table_hbm.at[ib], rows.at[pl.ds(ch, ch)], sem)
        c1.wait()
        c2.wait()
        pltpu.sync_copy(rows, out_hbm.at[pl.ds(w * bpw, bpw)])

    return gk(codebook, idx2)


def kernel(z, codebook):
    B, D, H, W = z.shape
    N = B * H * W
    M = codebook.shape[0]
    G = N // BLK

    z_flat = jnp.transpose(z, (0, 2, 3, 1)).reshape(-1, D)
    z_sq = jnp.sum(z_flat * z_flat, axis=1, keepdims=True)
    c_sq = jnp.sum(codebook * codebook, axis=1)[None, :]

    idx3, d2min3 = pl.pallas_call(
        _vq_body,
        grid=(G,),
        in_specs=[
            pl.BlockSpec((BLK, D), lambda i: (i, 0)),
            pl.BlockSpec((M, D), lambda i: (0, 0)),
            pl.BlockSpec((1, 1, BLK), lambda i: (i, 0, 0)),
            pl.BlockSpec((1, M), lambda i: (0, 0)),
        ],
        out_specs=[
            pl.BlockSpec((1, 1, BLK), lambda i: (i, 0, 0)),
            pl.BlockSpec((1, 1, BLK), lambda i: (i, 0, 0)),
        ],
        out_shape=[
            jax.ShapeDtypeStruct((G, 1, BLK), jnp.int32),
            jax.ShapeDtypeStruct((G, 1, BLK), jnp.float32),
        ],
    )(z_flat, codebook, z_sq.reshape(G, 1, BLK), c_sq)

    indices = idx3.reshape(N)
    q = _sc_gather(codebook, indices)
    vq_loss = (1.0 + BETA) * (jnp.sum(d2min3) / (N * D))
    q_st = z_flat + (q - z_flat)
    z_q = jnp.transpose(q_st.reshape(B, H, W, D), (0, 3, 1, 2))
    return z_q, vq_loss, indices.reshape(B, H, W)


# speedup vs baseline: 1.0098x; 1.0098x over previous
import functools

import jax
import jax.numpy as jnp
from jax import lax
from jax.experimental import pallas as pl
from jax.experimental.pallas import tpu as pltpu
from jax.experimental.pallas import tpu_sc as plsc

NUM_CODES = 1024
EMBED_DIM = 256
BETA = 0.25
BLK = 512


def _vq_body(z_ref, c2_ref, zsq_ref, csq_ref, idx_ref, d2_ref):
    zb = z_ref[...]
    cb2 = c2_ref[...]
    s2 = lax.dot_general(zb, cb2, (((1,), (1,)), ((), ())))
    zsq = zsq_ref[0, 0, :].reshape(BLK, 1)
    csq = csq_ref[0, :].reshape(1, NUM_CODES)
    d2 = (zsq + csq) - s2
    d2 = jnp.maximum(d2, 0.0)
    dist = jnp.sqrt(d2)
    minval = jnp.min(dist, axis=1, keepdims=True)
    iota = lax.broadcasted_iota(jnp.int32, (BLK, NUM_CODES), 1).astype(
        jnp.float32)
    idxf = jnp.min(jnp.where(dist == minval, iota, jnp.float32(NUM_CODES)),
                   axis=1)
    idx_ref[0, 0, :] = idxf.astype(jnp.int32)
    d2_ref[0, 0, :] = (minval * minval)[:, 0]


def _sc_gather(codebook, indices):
    info = plsc.get_sparse_core_info()
    nc, ns = info.num_cores, info.num_subcores
    nw = nc * ns
    n, d = indices.shape[0], codebook.shape[1]
    bpw = n // nw
    ch = 128
    nch = bpw // ch
    idx2 = indices.reshape(n // ch, ch)
    mesh = plsc.VectorSubcoreMesh(core_axis_name="c", subcore_axis_name="s")

    @functools.partial(
        pl.kernel, mesh=mesh,
        out_type=jax.ShapeDtypeStruct((n, d), jnp.float32),
        scratch_types=[
            pltpu.VMEM((ch,), jnp.int32),
            pltpu.VMEM((ch,), jnp.int32),
            pltpu.VMEM((bpw, d), jnp.float32),
            pltpu.SemaphoreType.DMA,
        ],
    )
    def gk(table_hbm, idx_hbm, out_hbm, ia, ib, rows, sem):
        w = lax.axis_index("s") * nc + lax.axis_index("c")
        r0 = w * nch
        pltpu.sync_copy(idx_hbm.at[r0], ia)
        pltpu.sync_copy(idx_hbm.at[r0 + 1], ib)
        c1 = pltpu.async_copy(table_hbm.at[ia], rows.at[pl.ds(0, ch)], sem)
        c2 = pltpu.async_copy(table_hbm.at[ib], rows.at[pl.ds(ch, ch)], sem)
        c1.wait()
        c2.wait()
        pltpu.sync_copy(rows, out_hbm.at[pl.ds(w * bpw, bpw)])

    return gk(codebook, idx2)


def kernel(z, codebook):
    B, D, H, W = z.shape
    N = B * H * W
    M = codebook.shape[0]
    G = N // BLK

    z_flat = jnp.transpose(z, (0, 2, 3, 1)).reshape(-1, D)
    z_sq = jnp.sum(z_flat * z_flat, axis=1, keepdims=True)
    c_sq = jnp.sum(codebook * codebook, axis=1)[None, :]

    idx3, d2min3 = pl.pallas_call(
        _vq_body,
        grid=(G,),
        in_specs=[
            pl.BlockSpec((BLK, D), lambda i: (i, 0)),
            pl.BlockSpec((M, D), lambda i: (0, 0)),
            pl.BlockSpec((1, 1, BLK), lambda i: (i, 0, 0)),
            pl.BlockSpec((1, M), lambda i: (0, 0)),
        ],
        out_specs=[
            pl.BlockSpec((1, 1, BLK), lambda i: (i, 0, 0)),
            pl.BlockSpec((1, 1, BLK), lambda i: (i, 0, 0)),
        ],
        out_shape=[
            jax.ShapeDtypeStruct((G, 1, BLK), jnp.int32),
            jax.ShapeDtypeStruct((G, 1, BLK), jnp.float32),
        ],
    )(z_flat, codebook + codebook, z_sq.reshape(G, 1, BLK), c_sq)

    indices = idx3.reshape(N)
    q = _sc_gather(codebook, indices)
    vq_loss = (1.0 + BETA) * (jnp.sum(d2min3) / (N * D))
    q_st = z_flat + (q - z_flat)
    z_q = jnp.transpose(q_st.reshape(B, H, W, D), (0, 3, 1, 2))
    return z_q, vq_loss, indices.reshape(B, H, W)
